# baseline (speedup 1.0000x reference)
import jax
import jax.numpy as jnp
from jax.experimental import pallas as pl

_K = 8192
_D = 256
_BM = 256
_CHUNK = 2736


def _chunk_min_argmin(dist, lo, hi):
    sl = dist[lo:hi, :]
    m = jnp.min(sl, axis=0, keepdims=True)
    iota = jax.lax.broadcasted_iota(jnp.int32, sl.shape, 0) + lo
    i = jnp.min(jnp.where(sl == m, iota, _K), axis=0, keepdims=True)
    return m, i


def _bf16(v):
    return v.astype(jnp.bfloat16).astype(jnp.float32)


def _vq_body(xt_ref, cb_ref, cbsq_ref, insq_ref, out_ref):
    xt = xt_ref[...]
    cb = cb_ref[...]
    mm = jnp.dot(cb.astype(jnp.bfloat16), xt.astype(jnp.bfloat16),
                 preferred_element_type=jnp.float32)
    insq = insq_ref[0, 0, :][None, :]
    cb_sqr = cbsq_ref[...]
    dist = (cb_sqr + insq) - 2.0 * mm
    m0, i0 = _chunk_min_argmin(dist, 0, _CHUNK)
    m1, i1 = _chunk_min_argmin(dist, _CHUNK, 2 * _CHUNK)
    m2, i2 = _chunk_min_argmin(dist, 2 * _CHUNK, _K)
    w1 = m1 < _bf16(m0)
    v = jnp.where(w1, m1, _bf16(m0))
    i = jnp.where(w1, i1, i0)
    w2 = m2 < _bf16(v)
    i = jnp.where(w2, i2, i)
    out_ref[0, 0, :] = i[0, :]


def kernel(z_e, codebook):
    lead_shape = z_e.shape[:-1]
    x = z_e.reshape(-1, _D)
    m = x.shape[0]
    nblk = m // _BM
    xt = x.T
    insq = jnp.sum(x * x, axis=1).reshape(nblk, 1, _BM)
    cbsq = jnp.sum(codebook * codebook, axis=1, keepdims=True)
    out = pl.pallas_call(
        _vq_body,
        grid=(nblk,),
        in_specs=[
            pl.BlockSpec((_D, _BM), lambda i: (0, i)),
            pl.BlockSpec((_K, _D), lambda i: (0, 0)),
            pl.BlockSpec((_K, 1), lambda i: (0, 0)),
            pl.BlockSpec((1, 1, _BM), lambda i: (i, 0, 0)),
        ],
        out_specs=pl.BlockSpec((1, 1, _BM), lambda i: (i, 0, 0)),
        out_shape=jax.ShapeDtypeStruct((nblk, 1, _BM), jnp.int32),
    )(xt, codebook, cbsq, insq)
    return out.reshape(lead_shape)


# speedup vs baseline: 1.1344x; 1.1344x over previous
import jax
import jax.numpy as jnp
from jax.experimental import pallas as pl

_K = 8192
_D = 256
_BM = 512
_CHUNK = 2736


def _chunk_min_argmin(dist, lo, hi):
    sl = dist[lo:hi, :]
    m = jnp.min(sl, axis=0, keepdims=True)
    iota = jax.lax.broadcasted_iota(jnp.int32, sl.shape, 0) + lo
    i = jnp.min(jnp.where(sl == m, iota, _K), axis=0, keepdims=True)
    return m, i


def _bf16(v):
    return v.astype(jnp.bfloat16).astype(jnp.float32)


def _vq_body(xt_ref, cb_ref, cbsq_ref, insq_ref, out_ref):
    xt = xt_ref[...]
    cb = cb_ref[...]
    mm = jnp.dot(cb.astype(jnp.bfloat16), xt.astype(jnp.bfloat16),
                 preferred_element_type=jnp.float32)
    insq = insq_ref[0, 0, :][None, :]
    cb_sqr = cbsq_ref[...]
    dist = (cb_sqr + insq) - 2.0 * mm
    m0, i0 = _chunk_min_argmin(dist, 0, _CHUNK)
    m1, i1 = _chunk_min_argmin(dist, _CHUNK, 2 * _CHUNK)
    m2, i2 = _chunk_min_argmin(dist, 2 * _CHUNK, _K)
    w1 = m1 < _bf16(m0)
    v = jnp.where(w1, m1, _bf16(m0))
    i = jnp.where(w1, i1, i0)
    w2 = m2 < _bf16(v)
    i = jnp.where(w2, i2, i)
    out_ref[0, 0, :] = i[0, :]


def kernel(z_e, codebook):
    lead_shape = z_e.shape[:-1]
    x = z_e.reshape(-1, _D)
    m = x.shape[0]
    nblk = m // _BM
    xt = x.T
    insq = jnp.sum(x * x, axis=1).reshape(nblk, 1, _BM)
    cbsq = jnp.sum(codebook * codebook, axis=1, keepdims=True)
    out = pl.pallas_call(
        _vq_body,
        grid=(nblk,),
        in_specs=[
            pl.BlockSpec((_D, _BM), lambda i: (0, i)),
            pl.BlockSpec((_K, _D), lambda i: (0, 0)),
            pl.BlockSpec((_K, 1), lambda i: (0, 0)),
            pl.BlockSpec((1, 1, _BM), lambda i: (i, 0, 0)),
        ],
        out_specs=pl.BlockSpec((1, 1, _BM), lambda i: (i, 0, 0)),
        out_shape=jax.ShapeDtypeStruct((nblk, 1, _BM), jnp.int32),
    )(xt, codebook, cbsq, insq)
    return out.reshape(lead_shape)


# speedup vs baseline: 1.2537x; 1.1052x over previous
import jax
import jax.numpy as jnp
from jax.experimental import pallas as pl

_K = 8192
_D = 256
_BM = 1024
_CHUNK = 2736


def _chunk_min_argmin(dist, lo, hi):
    sl = dist[lo:hi, :]
    m = jnp.min(sl, axis=0, keepdims=True)
    iota = jax.lax.broadcasted_iota(jnp.int32, sl.shape, 0) + lo
    i = jnp.min(jnp.where(sl == m, iota, _K), axis=0, keepdims=True)
    return m, i


def _bf16(v):
    return v.astype(jnp.bfloat16).astype(jnp.float32)


def _vq_body(xt_ref, cb_ref, cbsq_ref, insq_ref, out_ref):
    xt = xt_ref[...]
    cb = cb_ref[...]
    mm = jnp.dot(cb.astype(jnp.bfloat16), xt.astype(jnp.bfloat16),
                 preferred_element_type=jnp.float32)
    insq = insq_ref[0, 0, :][None, :]
    cb_sqr = cbsq_ref[...]
    dist = (cb_sqr + insq) - 2.0 * mm
    m0, i0 = _chunk_min_argmin(dist, 0, _CHUNK)
    m1, i1 = _chunk_min_argmin(dist, _CHUNK, 2 * _CHUNK)
    m2, i2 = _chunk_min_argmin(dist, 2 * _CHUNK, _K)
    w1 = m1 < _bf16(m0)
    v = jnp.where(w1, m1, _bf16(m0))
    i = jnp.where(w1, i1, i0)
    w2 = m2 < _bf16(v)
    i = jnp.where(w2, i2, i)
    out_ref[0, 0, :] = i[0, :]


def kernel(z_e, codebook):
    lead_shape = z_e.shape[:-1]
    x = z_e.reshape(-1, _D)
    m = x.shape[0]
    nblk = m // _BM
    xt = x.T
    insq = jnp.sum(x * x, axis=1).reshape(nblk, 1, _BM)
    cbsq = jnp.sum(codebook * codebook, axis=1, keepdims=True)
    out = pl.pallas_call(
        _vq_body,
        grid=(nblk,),
        in_specs=[
            pl.BlockSpec((_D, _BM), lambda i: (0, i)),
            pl.BlockSpec((_K, _D), lambda i: (0, 0)),
            pl.BlockSpec((_K, 1), lambda i: (0, 0)),
            pl.BlockSpec((1, 1, _BM), lambda i: (i, 0, 0)),
        ],
        out_specs=pl.BlockSpec((1, 1, _BM), lambda i: (i, 0, 0)),
        out_shape=jax.ShapeDtypeStruct((nblk, 1, _BM), jnp.int32),
    )(xt, codebook, cbsq, insq)
    return out.reshape(lead_shape)
